# baseline (speedup 1.0000x reference)
import functools

import jax
import jax.numpy as jnp
from jax import lax
from jax.experimental import pallas as pl
from jax.experimental.pallas import tpu as pltpu
from jax.experimental.pallas import tpu_sc as plsc

NG = 10000
NE = 160000
B = 256
N = 3 * NG
NPAD = 30720
G3 = 3 * B
D = 64
ND = 128
NW = 32
EPW = 15360
EPAD = NW * EPW
CH = 128
NCH = EPW // CH
NH = NPAD // 2
ACC_H = 7808
HTRASH = 7700
RPTH = ACC_H // 16
RB = 1280
NBLK = NPAD // RB
HBLK = NH // RB
EB = 2048

_F32 = jnp.float32


def _full(shape):
    return pl.BlockSpec(shape, lambda i: tuple(0 for _ in shape))


def _mesh():
    return plsc.VectorSubcoreMesh(core_axis_name="c", subcore_axis_name="s")


def _gather_body(t_hbm, row_hbm, col_hbm, gs_hbm,
                 idxr, idxc, br0, br1, br2, bc0, bc1, bc2,
                 gr0, gr1, gr2, gc0, gc1, gc2, wr0, wr1, wr2):
    wid = lax.axis_index("s") * 2 + lax.axis_index("c")
    base = pl.multiple_of(wid * EPW, EPW)
    pltpu.sync_copy(row_hbm.at[wid], idxr)
    pltpu.sync_copy(col_hbm.at[wid], idxc)
    bufr = (br0, br1, br2)
    bufc = (bc0, bc1, bc2)
    gsr = (gr0, gr1, gr2)
    gsc = (gc0, gc1, gc2)
    wsr = (wr0, wr1, wr2)

    def g_start(g, b):
        pltpu.async_copy(t_hbm.at[idxr.at[g]], bufr[b], gsr[b])
        pltpu.async_copy(t_hbm.at[idxc.at[g]], bufc[b], gsc[b])

    def g_wait(g, b):
        pltpu.make_async_copy(t_hbm.at[idxr.at[g]], bufr[b], gsr[b]).wait()
        pltpu.make_async_copy(t_hbm.at[idxc.at[g]], bufc[b], gsc[b]).wait()

    def add_halves(b):
        br = bufr[b]
        bc = bufc[b]

        def rowop(r, carry):
            for k in range(4):
                br[r, pl.ds(16 * k, 16)] = (br[r, pl.ds(16 * k, 16)]
                                            + bc[r, pl.ds(D + 16 * k, 16)])
            return carry

        lax.fori_loop(0, CH, rowop, 0)

    def w_start(g, b):
        off = base + g * CH
        pltpu.async_copy(bufr[b], gs_hbm.at[pl.ds(off, CH)], wsr[b])

    def w_wait(g, b):
        off = base + g * CH
        pltpu.make_async_copy(bufr[b], gs_hbm.at[pl.ds(off, CH)],
                              wsr[b]).wait()

    for b in range(2):
        g_start(b, b)

    def outer(i, carry):
        g0 = i * 3
        for b in range(3):
            g = g0 + b
            g_wait(g, b)
            add_halves(b)
            w_start(g, b)
            bp = (b + 2) % 3

            @pl.when(g >= 1)
            def _():
                w_wait(g - 1, bp)

            @pl.when(g + 2 < NCH)
            def _():
                g_start(g + 2, bp)
        return carry

    lax.fori_loop(0, NCH // 3, outer, 0)
    w_wait(NCH - 1, (NCH - 1) % 3)


@functools.cache
def _gather_kernel():
    return pl.kernel(
        _gather_body,
        out_type=jax.ShapeDtypeStruct((EPAD, ND), _F32),
        mesh=_mesh(),
        scratch_types=(
            [pltpu.VMEM((NCH, CH), jnp.int32)] * 2
            + [pltpu.VMEM((CH, ND), _F32)] * 6
            + [pltpu.SemaphoreType.DMA] * 9
        ),
    )


def _sc_gather_call(T, row2, col2):
    return _gather_kernel()(T, row2, col2)


def _zero_acc(z_hbm, acc, rbase):
    for j in range(3):
        pltpu.sync_copy(z_hbm, acc.at[pl.ds(rbase + j * CH, CH)])
    pltpu.sync_copy(z_hbm.at[pl.ds(0, RPTH - 3 * CH)],
                    acc.at[pl.ds(rbase + 3 * CH, RPTH - 3 * CH)])


def _scatter_body(hp_hbm, colm_hbm, z_hbm, out_hbm,
                  idxc, bf0, bf1, bf2, ls0, ls1, ls2, as0, as1, as2, acc):
    cc = lax.axis_index("c")
    ss = lax.axis_index("s")
    wid = ss * 2 + cc
    base = pl.multiple_of(wid * EPW, EPW)
    rbase = pl.multiple_of(ss * RPTH, RPTH)
    _zero_acc(z_hbm, acc, rbase)
    pltpu.sync_copy(colm_hbm.at[wid], idxc)
    plsc.subcore_barrier()
    bufs = (bf0, bf1, bf2)
    lsem = (ls0, ls1, ls2)
    asem = (as0, as1, as2)

    def l_start(g, b):
        pltpu.async_copy(hp_hbm.at[pl.ds(base + g * CH, CH)], bufs[b], lsem[b])

    def l_wait(g, b):
        pltpu.make_async_copy(hp_hbm.at[pl.ds(base + g * CH, CH)],
                              bufs[b], lsem[b]).wait()

    def a_start(g, b):
        pltpu.async_copy(bufs[b], acc.at[idxc.at[g]], asem[b], add=True)

    def a_wait(g, b):
        pltpu.make_async_copy(bufs[b], acc.at[idxc.at[g]], asem[b]).wait()

    for b in range(2):
        l_start(b, b)

    def outer(i, carry):
        g0 = i * 3
        for b in range(3):
            g = g0 + b
            l_wait(g, b)
            a_start(g, b)
            bp = (b + 2) % 3

            @pl.when(g >= 1)
            def _():
                a_wait(g - 1, bp)

            @pl.when(g + 2 < NCH)
            def _():
                l_start(g + 2, bp)
        return carry

    lax.fori_loop(0, NCH // 3, outer, 0)
    a_wait(NCH - 1, (NCH - 1) % 3)
    plsc.subcore_barrier()
    pltpu.sync_copy(acc.at[pl.ds(rbase, RPTH)],
                    out_hbm.at[cc, pl.ds(rbase, RPTH)])


@functools.cache
def _scatter_kernel():
    return pl.kernel(
        _scatter_body,
        out_type=jax.ShapeDtypeStruct((2, ACC_H, ND), _F32),
        mesh=_mesh(),
        scratch_types=(
            [pltpu.VMEM((NCH, CH), jnp.int32)]
            + [pltpu.VMEM((CH, ND), _F32)] * 3
            + [pltpu.SemaphoreType.DMA] * 6
            + [pltpu.VMEM_SHARED((ACC_H, ND), _F32)]
        ),
    )


def _sc_scatter_call(hp, sidxA, sidxB, zrow):
    sa = _scatter_kernel()(hp, sidxA, zrow)
    sb = _scatter_kernel()(hp, sidxB, zrow)
    return jnp.stack([sa, sb])


def _counts_body(pk_hbm, z_hbm, ones_hbm, out_hbm,
                 pkb, i00, i01, i10, i11, s00, s01, s10, s11, ones_v, acc):
    cc = lax.axis_index("c")
    ss = lax.axis_index("s")
    wid = ss * 2 + cc
    rbase = pl.multiple_of(ss * RPTH, RPTH)
    _zero_acc(z_hbm, acc, rbase)
    pltpu.sync_copy(ones_hbm, ones_v)
    pltpu.sync_copy(pk_hbm.at[wid], pkb)
    plsc.subcore_barrier()
    idx0 = (i00, i01)
    idx1 = (i10, i11)
    sm0 = (s00, s01)
    sm1 = (s10, s11)

    def a_start(b):
        pltpu.async_copy(ones_v.at[0], acc.at[idx0[b]], sm0[b], add=True)
        pltpu.async_copy(ones_v.at[1], acc.at[idx1[b]], sm1[b], add=True)

    def a_wait(b):
        pltpu.make_async_copy(ones_v.at[0], acc.at[idx0[b]], sm0[b]).wait()
        pltpu.make_async_copy(ones_v.at[1], acc.at[idx1[b]], sm1[b]).wait()

    def outer(i, carry):
        g0 = i * 2
        for b in range(2):
            g = g0 + b

            @pl.when(g >= 2)
            def _():
                a_wait(b)

            for v in range(CH // 16):
                x = pkb[g, pl.ds(16 * v, 16)]
                idx0[b][pl.ds(16 * v, 16)] = lax.bitwise_and(x, 8191)
                idx1[b][pl.ds(16 * v, 16)] = lax.shift_right_logical(x, 13)
            a_start(b)
        return carry

    lax.fori_loop(0, NCH // 2, outer, 0)
    for b in range(2):
        a_wait(b)
    plsc.subcore_barrier()
    pltpu.sync_copy(acc.at[pl.ds(rbase, RPTH)],
                    out_hbm.at[cc, pl.ds(rbase, RPTH)])


@functools.cache
def _counts_kernel():
    return pl.kernel(
        _counts_body,
        out_type=jax.ShapeDtypeStruct((2, ACC_H, ND), _F32),
        mesh=_mesh(),
        scratch_types=(
            [pltpu.VMEM((NCH, CH), jnp.int32)]
            + [pltpu.VMEM((CH,), jnp.int32)] * 4
            + [pltpu.SemaphoreType.DMA] * 4
            + [pltpu.VMEM((2, CH, ND), _F32)]
            + [pltpu.VMEM_SHARED((ACC_H, ND), _F32)]
        ),
    )


def _sc_counts_call(pkA, pkB, zrow, ones2):
    ca = _counts_kernel()(pkA, zrow, ones2)
    cb = _counts_kernel()(pkB, zrow, ones2)
    return jnp.stack([ca, cb])


def _enc_body(x, w1, b1, w2, b2, wa, wb, ex_ref, t_ref):
    t = jnp.maximum(x[...] @ w1[...] + b1[...], 0.0)
    ex = t @ w2[...] + b2[...]
    ex_ref[...] = ex
    t_ref[...] = jnp.concatenate([ex @ wa[...], ex @ wb[...]], axis=1)


def _encode(x, w1, b1, w2, b2, wa, wb):
    nblk = pl.BlockSpec((RB, ND), lambda i: (i, 0))
    return pl.pallas_call(
        _enc_body,
        grid=(NBLK,),
        in_specs=[nblk, _full((ND, ND)), _full((1, ND)), _full((ND, ND)),
                  _full((1, ND)), _full((ND, D)), _full((ND, D))],
        out_specs=[nblk, nblk],
        out_shape=[jax.ShapeDtypeStruct((NPAD, ND), _F32),
                   jax.ShapeDtypeStruct((NPAD, ND), _F32)],
    )(x, w1, b1, w2, b2, wa, wb)


def _edge0_body(gs, ear, colb, we1, be1, we2, be2, w1c, b1e, hp_ref):
    ee = jnp.maximum(ear[...] @ we1[...] + be1[...], 0.0) @ we2[...] + be2[...]
    h = jnp.maximum(gs[:, 0:D] + ee @ w1c[...] + b1e[...], 0.0)
    m = (colb[...] < NH).astype(_F32)
    hp_ref[...] = jnp.concatenate([h * m, h - h * m], axis=1)


def _edge0(gs, ear, colv, we1, be1, we2, be2, w1c, b1e):
    eblk = pl.BlockSpec((EB, ND), lambda i: (i, 0))
    return pl.pallas_call(
        _edge0_body,
        grid=(EPAD // EB,),
        in_specs=[eblk, pl.BlockSpec((EB, 4), lambda i: (i, 0)),
                  pl.BlockSpec((EB, 1), lambda i: (i, 0)),
                  _full((4, D)), _full((1, D)), _full((D, D)), _full((1, D)),
                  _full((D, D)), _full((1, D))],
        out_specs=eblk,
        out_shape=jax.ShapeDtypeStruct((EPAD, ND), _F32),
    )(gs, ear, colv, we1, be1, we2, be2, w1c, b1e)


def _edge1_body(gs, hp0, colb, w1c, b1e, w2e, b2e, hp_ref):
    h0 = hp0[:, 0:D] + hp0[:, D:ND]
    w2c = w2e[...] @ w1c[...]
    bias = b2e[...] @ w1c[...] + b1e[...]
    h = jnp.maximum(gs[:, 0:D] + h0 @ w2c + bias, 0.0)
    m = (colb[...] < NH).astype(_F32)
    hp_ref[...] = jnp.concatenate([h * m, h - h * m], axis=1)


def _edge1(gs, hp0, colv, w1c, b1e, w2e, b2e):
    eblk = pl.BlockSpec((EB, ND), lambda i: (i, 0))
    return pl.pallas_call(
        _edge1_body,
        grid=(EPAD // EB,),
        in_specs=[eblk, eblk, pl.BlockSpec((EB, 1), lambda i: (i, 0)),
                  _full((D, D)), _full((1, D)), _full((D, D)), _full((1, D))],
        out_specs=eblk,
        out_shape=jax.ShapeDtypeStruct((EPAD, ND), _F32),
    )(gs, hp0, colv, w1c, b1e, w2e, b2e)


def _node_common(i, s2, cnt2, x, btc, w2e, b2e, wn1x, wn1a, bn1, wn2, bn2):
    s3 = s2[...][0]
    c3 = cnt2[...][0]
    sfull = s3[0] + s3[1]
    cfull = c3[0] + c3[1]
    lo = i < HBLK
    s = jnp.where(lo, sfull[:, 0:D], sfull[:, D:ND])
    c = jnp.where(lo, cfull[:, 0:1], cfull[:, D:D + 1])
    hs = s / jnp.maximum(c, 1.0)
    agg = hs @ w2e[...] + jnp.where(c > 0, 1.0, 0.0) * b2e[...]
    z = jnp.maximum(x[...] @ wn1x[...] + agg @ wn1a[...] + bn1[...], 0.0)
    x2 = z @ wn2[...] + bn2[...]
    ids = btc[0, 0, :]
    m = (lax.broadcasted_iota(jnp.int32, (G3, RB), 0)
         == ids[None, :]).astype(_F32)
    return x2, m


def _glob_mlp(uin, nacc, nb, wg1u, wg1n, bg1, wg2, bg2):
    nmean = nacc[...] / jnp.maximum(nb[...], 1.0)
    g = jnp.maximum(uin @ wg1u[...] + nmean @ wg1n[...] + bg1[...], 0.0)
    return g @ wg2[...] + bg2[...]


def _node0_body(s2, cnt2, x, btc, w2e, b2e, wn1x, wn1a, bn1, wn2, bn2,
                wa, wb, wg1u, wg1n, bg1, wg2, bg2,
                x2_ref, t_ref, u_ref, nacc, nb):
    i = pl.program_id(0)

    @pl.when(i == 0)
    def _():
        nacc[...] = jnp.zeros_like(nacc)
        nb[...] = jnp.zeros_like(nb)

    x2, m = _node_common(i, s2, cnt2, x, btc, w2e, b2e, wn1x, wn1a, bn1,
                         wn2, bn2)
    x2_ref[...] = x2
    t_ref[...] = jnp.concatenate([x2 @ wa[...], x2 @ wb[...]], axis=1)
    nacc[...] += m @ x2
    nb[...] += jnp.sum(m, axis=1, keepdims=True)

    @pl.when(i == NBLK - 1)
    def _():
        uin = jnp.full((G3, D), 0.1, _F32)
        u_ref[...] = _glob_mlp(uin, nacc, nb, wg1u, wg1n, bg1, wg2, bg2)


def _node0(s2, cnt2, x, btc, w2e, b2e, wn1x, wn1a, bn1, wn2, bn2,
           wa, wb, wg1u, wg1n, bg1, wg2, bg2):
    nblk = pl.BlockSpec((RB, ND), lambda i: (i, 0))
    return pl.pallas_call(
        _node0_body,
        grid=(NBLK,),
        in_specs=[pl.BlockSpec((1, 2, RB, ND),
                                lambda i: ((i % HBLK) // 6, 0, (i % HBLK) % 6, 0)),
                  pl.BlockSpec((1, 2, RB, ND),
                                lambda i: ((i % HBLK) // 6, 0, (i % HBLK) % 6, 0)),
                  nblk,
                  pl.BlockSpec((1, 1, RB), lambda i: (i, 0, 0)),
                  _full((D, D)), _full((1, D)),
                  _full((ND, ND)), _full((D, ND)), _full((1, ND)),
                  _full((ND, ND)), _full((1, ND)),
                  _full((ND, D)), _full((ND, D)),
                  _full((D, D)), _full((ND, D)), _full((1, D)),
                  _full((D, D)), _full((1, D))],
        out_specs=[nblk, nblk, pl.BlockSpec((G3, D), lambda i: (0, 0))],
        out_shape=[jax.ShapeDtypeStruct((NPAD, ND), _F32),
                   jax.ShapeDtypeStruct((NPAD, ND), _F32),
                   jax.ShapeDtypeStruct((G3, D), _F32)],
        scratch_shapes=[pltpu.VMEM((G3, ND), _F32),
                        pltpu.VMEM((G3, 1), _F32)],
    )(s2, cnt2, x, btc, w2e, b2e, wn1x, wn1a, bn1, wn2, bn2,
      wa, wb, wg1u, wg1n, bg1, wg2, bg2)


def _node1_body(s2, cnt2, x, btc, uin, w2e, b2e, wn1x, wn1a, bn1, wn2, bn2,
                wg1u, wg1n, bg1, wg2, bg2, u_ref, nacc, nb):
    i = pl.program_id(0)

    @pl.when(i == 0)
    def _():
        nacc[...] = jnp.zeros_like(nacc)
        nb[...] = jnp.zeros_like(nb)

    x2, m = _node_common(i, s2, cnt2, x, btc, w2e, b2e, wn1x, wn1a, bn1,
                         wn2, bn2)
    nacc[...] += m @ x2
    nb[...] += jnp.sum(m, axis=1, keepdims=True)

    @pl.when(i == NBLK - 1)
    def _():
        u_ref[...] = _glob_mlp(uin[...], nacc, nb, wg1u, wg1n, bg1, wg2, bg2)


def _node1(s2, cnt2, x, btc, uin, w2e, b2e, wn1x, wn1a, bn1, wn2, bn2,
           wg1u, wg1n, bg1, wg2, bg2):
    nblk = pl.BlockSpec((RB, ND), lambda i: (i, 0))
    return pl.pallas_call(
        _node1_body,
        grid=(NBLK,),
        in_specs=[pl.BlockSpec((1, 2, RB, ND),
                                lambda i: ((i % HBLK) // 6, 0, (i % HBLK) % 6, 0)),
                  pl.BlockSpec((1, 2, RB, ND),
                                lambda i: ((i % HBLK) // 6, 0, (i % HBLK) % 6, 0)),
                  nblk,
                  pl.BlockSpec((1, 1, RB), lambda i: (i, 0, 0)),
                  pl.BlockSpec((G3, D), lambda i: (0, 0)),
                  _full((D, D)), _full((1, D)),
                  _full((ND, ND)), _full((D, ND)), _full((1, ND)),
                  _full((ND, ND)), _full((1, ND)),
                  _full((D, D)), _full((ND, D)), _full((1, D)),
                  _full((D, D)), _full((1, D))],
        out_specs=pl.BlockSpec((G3, D), lambda i: (0, 0)),
        out_shape=jax.ShapeDtypeStruct((G3, D), _F32),
        scratch_shapes=[pltpu.VMEM((G3, ND), _F32),
                        pltpu.VMEM((G3, 1), _F32)],
    )(s2, cnt2, x, btc, uin, w2e, b2e, wn1x, wn1a, bn1, wn2, bn2,
      wg1u, wg1n, bg1, wg2, bg2)


def _p2_body(u, tc, y1, y2, ym,
             en1w, en1b, en2w, en2b, ee1w, ee1b, ee2w, ee2b,
             ew1, eb1, ew2, eb2, nw1, nb1, nw2, nb2,
             gw1, gb1, gw2, gb2, lw1, lb1, lw2, lb2, out_ref):
    x = (jnp.maximum(u[...] @ en1w[...] + en1b[...], 0.0) @ en2w[...]
         + en2b[...])
    t = tc[...]
    r1 = y1[...] / ym[...]
    r2 = y2[...] / ym[...]

    def enc_edge(r):
        h = jnp.maximum(t @ ee1w[0:1, :] + r @ ee1w[1:2, :] + ee1b[...], 0.0)
        return h @ ee2w[...] + ee2b[...]

    c1 = enc_edge(r1)
    c2 = enc_edge(r2)
    ea = jnp.concatenate([c1, c1, c2, c2], axis=0)
    ug = jnp.full((B, 32), 0.1, _F32)
    for _ in range(2):
        xa, xb, xc = x[0:B], x[B:2 * B], x[2 * B:3 * B]
        src = jnp.concatenate([xa, xc, xb, xc], axis=0)
        dst = jnp.concatenate([xc, xa, xc, xb], axis=0)
        cat = jnp.concatenate([src, dst, ea], axis=1)
        eh = jnp.maximum(cat @ ew1[...] + eb1[...], 0.0)
        ea = eh @ ew2[...] + eb2[...]
        agg = jnp.concatenate(
            [ea[B:2 * B], ea[3 * B:4 * B],
             (ea[0:B] + ea[2 * B:3 * B]) * 0.5], axis=0)
        ncat = jnp.concatenate([x, agg], axis=1)
        x = (jnp.maximum(ncat @ nw1[...] + nb1[...], 0.0) @ nw2[...]
             + nb2[...])
        xa, xb, xc = x[0:B], x[B:2 * B], x[2 * B:3 * B]
        nmean = (xa + xb + xc) * (1.0 / 3.0)
        gcat = jnp.concatenate([ug, nmean], axis=1)
        ug = (jnp.maximum(gcat @ gw1[...] + gb1[...], 0.0) @ gw2[...]
              + gb2[...])
    out_ref[...] = (jnp.maximum(ug @ lw1[...] + lb1[...], 0.0) @ lw2[...]
                    + lb2[...])


def _phase2(u, tc, y1, y2, ym, wlist):
    specs = [_full((G3, D))] + [_full((B, 1))] * 4
    specs += [_full(w.shape) for w in wlist]
    return pl.pallas_call(
        _p2_body,
        grid=(1,),
        in_specs=specs,
        out_specs=_full((B, 15)),
        out_shape=jax.ShapeDtypeStruct((B, 15), _F32),
    )(u, tc, y1, y2, ym, *wlist)


def kernel(x_p1, ei_p1, ea_p1, y_p1, btc_p1, x_p2, ei_p2, ea_p2, y_p2, btc_p2,
           x_pm, ei_pm, ea_pm, y_pm, btc_pm, Temperature, params):
    x_all = jnp.concatenate(
        [x_p1, x_p2, x_pm, jnp.zeros((NPAD - N, ND), _F32)], axis=0)
    row = jnp.concatenate([ei_p1[0], ei_p2[0] + NG, ei_pm[0] + 2 * NG,
                           jnp.zeros((EPAD - 3 * NE,), jnp.int32)])
    col = jnp.concatenate([ei_p1[1], ei_p2[1] + NG, ei_pm[1] + 2 * NG,
                           jnp.full((EPAD - 3 * NE,), N, jnp.int32)])
    row2 = row.reshape(NW, NCH, CH)
    col2 = col.reshape(NW, NCH, CH)
    colm = col % NH
    haH = NH // 2
    sidxA = jnp.where(colm < haH, colm, HTRASH).reshape(NW, NCH, CH)
    sidxB = jnp.where(colm >= haH, colm - haH, HTRASH).reshape(NW, NCH, CH)
    p0 = col < NH
    inA = colm < haH
    cA0 = jnp.where(p0 & inA, colm, HTRASH)
    cA1 = jnp.where(~p0 & inA, colm, HTRASH)
    cB0 = jnp.where(p0 & ~inA, colm - haH, HTRASH)
    cB1 = jnp.where(~p0 & ~inA, colm - haH, HTRASH)
    pkA = (cA0 | (cA1 << 13)).reshape(NW, NCH, CH)
    pkB = (cB0 | (cB1 << 13)).reshape(NW, NCH, CH)
    colv = col.reshape(EPAD, 1)
    ea_all = jnp.concatenate(
        [ea_p1, ea_p2, ea_pm, jnp.zeros((EPAD - 3 * NE, 4), _F32)], axis=0)
    btc = jnp.concatenate([btc_p1, btc_p2 + B, btc_pm + 2 * B,
                           jnp.full((NPAD - N,), G3, jnp.int32)])
    btc = btc.reshape(NBLK, 1, RB)
    zrow = jnp.zeros((CH, ND), _F32)
    o1 = jnp.ones((CH, D), _F32)
    o0 = jnp.zeros((CH, D), _F32)
    ones2 = jnp.stack([jnp.concatenate([o1, o0], axis=1),
                       jnp.concatenate([o0, o1], axis=1)])

    def wb(p):
        return [q for (W, bias) in p for q in (W, bias.reshape(1, -1))]

    en1w1, en1b1, en1w2, en1b2 = wb(params['enc_node_1'])
    ee1w1, ee1b1, ee1w2, ee1b2 = wb(params['enc_edge_1'])
    e1w1, e1b1, e1w2, e1b2 = wb(params['edge1'])
    w1a, w1b, w1c = e1w1[0:ND], e1w1[ND:2 * ND], e1w1[2 * ND:2 * ND + D]
    n1w1, n1b1, n1w2, n1b2 = wb(params['node1'])
    wn1x, wn1a = n1w1[0:ND], n1w1[ND:ND + D]
    g1w1, g1b1, g1w2, g1b2 = wb(params['glob1'])
    wg1u, wg1n = g1w1[0:D], g1w1[D:D + ND]
    p2w = (wb(params['enc_node_2']) + wb(params['enc_edge_2'])
           + wb(params['edge2']) + wb(params['node2'])
           + wb(params['glob2']) + wb(params['last']))

    ex, T = _encode(x_all, en1w1, en1b1, en1w2, en1b2, w1a, w1b)
    cnt2 = _sc_counts_call(pkA, pkB, zrow, ones2)

    gS = _sc_gather_call(T, row2, col2)
    hp0 = _edge0(gS, ea_all, colv, ee1w1, ee1b1, ee1w2, ee1b2, w1c, e1b1)
    S2 = _sc_scatter_call(hp0, sidxA, sidxB, zrow)
    x2, T2, u1 = _node0(S2, cnt2, ex, btc, e1w2, e1b2,
                        wn1x, wn1a, n1b1, n1w2, n1b2, w1a, w1b,
                        wg1u, wg1n, g1b1, g1w2, g1b2)

    gS = _sc_gather_call(T2, row2, col2)
    hp1 = _edge1(gS, hp0, colv, w1c, e1b1, e1w2, e1b2)
    S2 = _sc_scatter_call(hp1, sidxA, sidxB, zrow)
    u2 = _node1(S2, cnt2, x2, btc, u1, e1w2, e1b2,
                wn1x, wn1a, n1b1, n1w2, n1b2,
                wg1u, wg1n, g1b1, g1w2, g1b2)

    return _phase2(u2, Temperature.reshape(B, 1), y_p1.reshape(B, 1),
                   y_p2.reshape(B, 1), y_pm.reshape(B, 1), p2w)


# speedup vs baseline: 1.2097x; 1.2097x over previous
import functools

import jax
import jax.numpy as jnp
from jax import lax
from jax.experimental import pallas as pl
from jax.experimental.pallas import tpu as pltpu
from jax.experimental.pallas import tpu_sc as plsc

NG = 10000
NE = 160000
B = 256
N = 3 * NG
NPAD = 30720
G3 = 3 * B
D = 64
ND = 128
NW = 32
EPW = 15360
EPAD = NW * EPW
CH = 128
NCH = EPW // CH
NH = NPAD // 2
ACC_H = 7808
HTRASH = 7700
RPTH = ACC_H // 16
RB = 1280
NBLK = NPAD // RB
HBLK = NH // RB
EB = 2048
CHS = 256
NCHS = EPW // CHS

_F32 = jnp.float32


def _full(shape):
    return pl.BlockSpec(shape, lambda i: tuple(0 for _ in shape))


def _mesh():
    return plsc.VectorSubcoreMesh(core_axis_name="c", subcore_axis_name="s")


def _gather_body(t_hbm, row_hbm, col_hbm, gs_hbm,
                 idxr, idxc, br0, br1, br2, bc0, bc1, bc2,
                 gr0, gr1, gr2, gc0, gc1, gc2, wr0, wr1, wr2):
    wid = lax.axis_index("s") * 2 + lax.axis_index("c")
    base = pl.multiple_of(wid * EPW, EPW)
    pltpu.sync_copy(row_hbm.at[wid], idxr)
    pltpu.sync_copy(col_hbm.at[wid], idxc)
    bufr = (br0, br1, br2)
    bufc = (bc0, bc1, bc2)
    gsr = (gr0, gr1, gr2)
    gsc = (gc0, gc1, gc2)
    wsr = (wr0, wr1, wr2)

    def g_start(g, b):
        pltpu.async_copy(t_hbm.at[idxr.at[g]], bufr[b], gsr[b])
        pltpu.async_copy(t_hbm.at[idxc.at[g]], bufc[b], gsc[b])

    def g_wait(g, b):
        pltpu.make_async_copy(t_hbm.at[idxr.at[g]], bufr[b], gsr[b]).wait()
        pltpu.make_async_copy(t_hbm.at[idxc.at[g]], bufc[b], gsc[b]).wait()

    def add_halves(b):
        br = bufr[b]
        bc = bufc[b]

        def rowop(r, carry):
            for k in range(4):
                br[r, pl.ds(16 * k, 16)] = (br[r, pl.ds(16 * k, 16)]
                                            + bc[r, pl.ds(D + 16 * k, 16)])
            return carry

        lax.fori_loop(0, CH, rowop, 0)

    def w_start(g, b):
        off = base + g * CH
        pltpu.async_copy(bufr[b], gs_hbm.at[pl.ds(off, CH)], wsr[b])

    def w_wait(g, b):
        off = base + g * CH
        pltpu.make_async_copy(bufr[b], gs_hbm.at[pl.ds(off, CH)],
                              wsr[b]).wait()

    for b in range(2):
        g_start(b, b)

    def outer(i, carry):
        g0 = i * 3
        for b in range(3):
            g = g0 + b
            g_wait(g, b)
            add_halves(b)
            w_start(g, b)
            bp = (b + 2) % 3

            @pl.when(g >= 1)
            def _():
                w_wait(g - 1, bp)

            @pl.when(g + 2 < NCH)
            def _():
                g_start(g + 2, bp)
        return carry

    lax.fori_loop(0, NCH // 3, outer, 0)
    w_wait(NCH - 1, (NCH - 1) % 3)


@functools.cache
def _gather_kernel():
    return pl.kernel(
        _gather_body,
        out_type=jax.ShapeDtypeStruct((EPAD, ND), _F32),
        mesh=_mesh(),
        scratch_types=(
            [pltpu.VMEM((NCH, CH), jnp.int32)] * 2
            + [pltpu.VMEM((CH, ND), _F32)] * 6
            + [pltpu.SemaphoreType.DMA] * 9
        ),
    )


def _sc_gather_call(T, row2, col2):
    return _gather_kernel()(T, row2, col2)


def _zero_acc(z_hbm, acc, rbase):
    for j in range(3):
        pltpu.sync_copy(z_hbm, acc.at[pl.ds(rbase + j * CH, CH)])
    pltpu.sync_copy(z_hbm.at[pl.ds(0, RPTH - 3 * CH)],
                    acc.at[pl.ds(rbase + 3 * CH, RPTH - 3 * CH)])


def _scatter_body(hp_hbm, colm_hbm, z_hbm, out_hbm,
                  idxc, bf0, bf1, bf2, ls0, ls1, ls2, as0, as1, as2, acc):
    cc = lax.axis_index("c")
    ss = lax.axis_index("s")
    wid = ss * 2 + cc
    base = pl.multiple_of(wid * EPW, EPW)
    rbase = pl.multiple_of(ss * RPTH, RPTH)
    _zero_acc(z_hbm, acc, rbase)
    pltpu.sync_copy(colm_hbm.at[wid], idxc)
    plsc.subcore_barrier()
    bufs = (bf0, bf1, bf2)
    lsem = (ls0, ls1, ls2)
    asem = (as0, as1, as2)

    def l_start(g, b):
        pltpu.async_copy(hp_hbm.at[pl.ds(base + g * CH, CH)], bufs[b], lsem[b])

    def l_wait(g, b):
        pltpu.make_async_copy(hp_hbm.at[pl.ds(base + g * CH, CH)],
                              bufs[b], lsem[b]).wait()

    def a_start(g, b):
        ix = plsc.Indices(idxc.at[g], ignored_value=HTRASH)
        pltpu.async_copy(bufs[b], acc.at[ix], asem[b], add=True)

    def a_wait(g, b):
        ix = plsc.Indices(idxc.at[g], ignored_value=HTRASH)
        pltpu.make_async_copy(bufs[b], acc.at[ix], asem[b]).wait()

    for b in range(2):
        l_start(b, b)

    def outer(i, carry):
        g0 = i * 3
        for b in range(3):
            g = g0 + b
            l_wait(g, b)
            a_start(g, b)
            bp = (b + 2) % 3

            @pl.when(g >= 1)
            def _():
                a_wait(g - 1, bp)

            @pl.when(g + 2 < NCH)
            def _():
                l_start(g + 2, bp)
        return carry

    lax.fori_loop(0, NCH // 3, outer, 0)
    a_wait(NCH - 1, (NCH - 1) % 3)
    plsc.subcore_barrier()
    pltpu.sync_copy(acc.at[pl.ds(rbase, RPTH)],
                    out_hbm.at[cc, pl.ds(rbase, RPTH)])


@functools.cache
def _scatter_kernel():
    return pl.kernel(
        _scatter_body,
        out_type=jax.ShapeDtypeStruct((2, ACC_H, ND), _F32),
        mesh=_mesh(),
        scratch_types=(
            [pltpu.VMEM((NCH, CH), jnp.int32)]
            + [pltpu.VMEM((CH, ND), _F32)] * 3
            + [pltpu.SemaphoreType.DMA] * 6
            + [pltpu.VMEM_SHARED((ACC_H, ND), _F32)]
        ),
    )


def _sc_scatter_call(hp, sidxA, sidxB, zrow):
    sa = _scatter_kernel()(hp, sidxA, zrow)
    sb = _scatter_kernel()(hp, sidxB, zrow)
    return jnp.stack([sa, sb])


def _counts_body(pk_hbm, z_hbm, ones_hbm, out_hbm,
                 pkb, i00, i01, i10, i11, s00, s01, s10, s11, ones_v, acc):
    cc = lax.axis_index("c")
    ss = lax.axis_index("s")
    wid = ss * 2 + cc
    rbase = pl.multiple_of(ss * RPTH, RPTH)
    _zero_acc(z_hbm, acc, rbase)
    pltpu.sync_copy(ones_hbm, ones_v)
    pltpu.sync_copy(pk_hbm.at[wid], pkb)
    plsc.subcore_barrier()
    idx0 = (i00, i01)
    idx1 = (i10, i11)
    sm0 = (s00, s01)
    sm1 = (s10, s11)

    def a_start(b):
        ix0 = plsc.Indices(idx0[b], ignored_value=HTRASH)
        ix1 = plsc.Indices(idx1[b], ignored_value=HTRASH)
        pltpu.async_copy(ones_v.at[0], acc.at[ix0], sm0[b], add=True)
        pltpu.async_copy(ones_v.at[1], acc.at[ix1], sm1[b], add=True)

    def a_wait(b):
        ix0 = plsc.Indices(idx0[b], ignored_value=HTRASH)
        ix1 = plsc.Indices(idx1[b], ignored_value=HTRASH)
        pltpu.make_async_copy(ones_v.at[0], acc.at[ix0], sm0[b]).wait()
        pltpu.make_async_copy(ones_v.at[1], acc.at[ix1], sm1[b]).wait()

    def outer(i, carry):
        g0 = i * 2
        for b in range(2):
            g = g0 + b

            @pl.when(g >= 2)
            def _():
                a_wait(b)

            for v in range(CH // 16):
                x = pkb[g, pl.ds(16 * v, 16)]
                idx0[b][pl.ds(16 * v, 16)] = lax.bitwise_and(x, 8191)
                idx1[b][pl.ds(16 * v, 16)] = lax.shift_right_logical(x, 13)
            a_start(b)
        return carry

    lax.fori_loop(0, NCH // 2, outer, 0)
    for b in range(2):
        a_wait(b)
    plsc.subcore_barrier()
    pltpu.sync_copy(acc.at[pl.ds(rbase, RPTH)],
                    out_hbm.at[cc, pl.ds(rbase, RPTH)])


@functools.cache
def _counts_kernel():
    return pl.kernel(
        _counts_body,
        out_type=jax.ShapeDtypeStruct((2, ACC_H, ND), _F32),
        mesh=_mesh(),
        scratch_types=(
            [pltpu.VMEM((NCH, CH), jnp.int32)]
            + [pltpu.VMEM((CH,), jnp.int32)] * 4
            + [pltpu.SemaphoreType.DMA] * 4
            + [pltpu.VMEM((2, CH, ND), _F32)]
            + [pltpu.VMEM_SHARED((ACC_H, ND), _F32)]
        ),
    )


def _sc_counts_call(pkA, pkB, zrow, ones2):
    ca = _counts_kernel()(pkA, zrow, ones2)
    cb = _counts_kernel()(pkB, zrow, ones2)
    return jnp.stack([ca, cb])


def _enc_body(x, w1, b1, w2, b2, wa, wb, ex_ref, t_ref):
    t = jnp.maximum(x[...] @ w1[...] + b1[...], 0.0)
    ex = t @ w2[...] + b2[...]
    ex_ref[...] = ex
    t_ref[...] = jnp.concatenate([ex @ wa[...], ex @ wb[...]], axis=1)


def _encode(x, w1, b1, w2, b2, wa, wb):
    nblk = pl.BlockSpec((RB, ND), lambda i: (i, 0))
    return pl.pallas_call(
        _enc_body,
        grid=(NBLK,),
        in_specs=[nblk, _full((ND, ND)), _full((1, ND)), _full((ND, ND)),
                  _full((1, ND)), _full((ND, D)), _full((ND, D))],
        out_specs=[nblk, nblk],
        out_shape=[jax.ShapeDtypeStruct((NPAD, ND), _F32),
                   jax.ShapeDtypeStruct((NPAD, ND), _F32)],
    )(x, w1, b1, w2, b2, wa, wb)


def _edge0_body(gs, ear, colb, we1, be1, we2, be2, w1c, b1e, hp_ref):
    ee = jnp.maximum(ear[...] @ we1[...] + be1[...], 0.0) @ we2[...] + be2[...]
    h = jnp.maximum(gs[:, 0:D] + ee @ w1c[...] + b1e[...], 0.0)
    m = (colb[...] < NH).astype(_F32)
    hp_ref[...] = jnp.concatenate([h * m, h - h * m], axis=1)


def _edge0(gs, ear, colv, we1, be1, we2, be2, w1c, b1e):
    eblk = pl.BlockSpec((EB, ND), lambda i: (i, 0))
    return pl.pallas_call(
        _edge0_body,
        grid=(EPAD // EB,),
        in_specs=[eblk, pl.BlockSpec((EB, 4), lambda i: (i, 0)),
                  pl.BlockSpec((EB, 1), lambda i: (i, 0)),
                  _full((4, D)), _full((1, D)), _full((D, D)), _full((1, D)),
                  _full((D, D)), _full((1, D))],
        out_specs=eblk,
        out_shape=jax.ShapeDtypeStruct((EPAD, ND), _F32),
    )(gs, ear, colv, we1, be1, we2, be2, w1c, b1e)


def _edge1_body(gs, hp0, colb, w1c, b1e, w2e, b2e, hp_ref):
    h0 = hp0[:, 0:D] + hp0[:, D:ND]
    w2c = w2e[...] @ w1c[...]
    bias = b2e[...] @ w1c[...] + b1e[...]
    h = jnp.maximum(gs[:, 0:D] + h0 @ w2c + bias, 0.0)
    m = (colb[...] < NH).astype(_F32)
    hp_ref[...] = jnp.concatenate([h * m, h - h * m], axis=1)


def _edge1(gs, hp0, colv, w1c, b1e, w2e, b2e):
    eblk = pl.BlockSpec((EB, ND), lambda i: (i, 0))
    return pl.pallas_call(
        _edge1_body,
        grid=(EPAD // EB,),
        in_specs=[eblk, eblk, pl.BlockSpec((EB, 1), lambda i: (i, 0)),
                  _full((D, D)), _full((1, D)), _full((D, D)), _full((1, D))],
        out_specs=eblk,
        out_shape=jax.ShapeDtypeStruct((EPAD, ND), _F32),
    )(gs, hp0, colv, w1c, b1e, w2e, b2e)


def _node_common(i, s2, cnt2, x, btc, w2e, b2e, wn1x, wn1a, bn1, wn2, bn2):
    s3 = s2[...][0]
    c3 = cnt2[...][0]
    sfull = s3[0] + s3[1]
    cfull = c3[0] + c3[1]
    lo = i < HBLK
    s = jnp.where(lo, sfull[:, 0:D], sfull[:, D:ND])
    c = jnp.where(lo, cfull[:, 0:1], cfull[:, D:D + 1])
    hs = s / jnp.maximum(c, 1.0)
    agg = hs @ w2e[...] + jnp.where(c > 0, 1.0, 0.0) * b2e[...]
    z = jnp.maximum(x[...] @ wn1x[...] + agg @ wn1a[...] + bn1[...], 0.0)
    x2 = z @ wn2[...] + bn2[...]
    ids = btc[0, 0, :]
    m = (lax.broadcasted_iota(jnp.int32, (G3, RB), 0)
         == ids[None, :]).astype(_F32)
    return x2, m


def _glob_mlp(uin, nacc, nb, wg1u, wg1n, bg1, wg2, bg2):
    nmean = nacc[...] / jnp.maximum(nb[...], 1.0)
    g = jnp.maximum(uin @ wg1u[...] + nmean @ wg1n[...] + bg1[...], 0.0)
    return g @ wg2[...] + bg2[...]


def _node0_body(s2, cnt2, x, btc, w2e, b2e, wn1x, wn1a, bn1, wn2, bn2,
                wa, wb, wg1u, wg1n, bg1, wg2, bg2,
                x2_ref, t_ref, u_ref, nacc, nb):
    i = pl.program_id(0)

    @pl.when(i == 0)
    def _():
        nacc[...] = jnp.zeros_like(nacc)
        nb[...] = jnp.zeros_like(nb)

    x2, m = _node_common(i, s2, cnt2, x, btc, w2e, b2e, wn1x, wn1a, bn1,
                         wn2, bn2)
    x2_ref[...] = x2
    t_ref[...] = jnp.concatenate([x2 @ wa[...], x2 @ wb[...]], axis=1)
    nacc[...] += m @ x2
    nb[...] += jnp.sum(m, axis=1, keepdims=True)

    @pl.when(i == NBLK - 1)
    def _():
        uin = jnp.full((G3, D), 0.1, _F32)
        u_ref[...] = _glob_mlp(uin, nacc, nb, wg1u, wg1n, bg1, wg2, bg2)


def _node0(s2, cnt2, x, btc, w2e, b2e, wn1x, wn1a, bn1, wn2, bn2,
           wa, wb, wg1u, wg1n, bg1, wg2, bg2):
    nblk = pl.BlockSpec((RB, ND), lambda i: (i, 0))
    return pl.pallas_call(
        _node0_body,
        grid=(NBLK,),
        in_specs=[pl.BlockSpec((1, 2, RB, ND),
                                lambda i: ((i % HBLK) // 6, 0, (i % HBLK) % 6, 0)),
                  pl.BlockSpec((1, 2, RB, ND),
                                lambda i: ((i % HBLK) // 6, 0, (i % HBLK) % 6, 0)),
                  nblk,
                  pl.BlockSpec((1, 1, RB), lambda i: (i, 0, 0)),
                  _full((D, D)), _full((1, D)),
                  _full((ND, ND)), _full((D, ND)), _full((1, ND)),
                  _full((ND, ND)), _full((1, ND)),
                  _full((ND, D)), _full((ND, D)),
                  _full((D, D)), _full((ND, D)), _full((1, D)),
                  _full((D, D)), _full((1, D))],
        out_specs=[nblk, nblk, pl.BlockSpec((G3, D), lambda i: (0, 0))],
        out_shape=[jax.ShapeDtypeStruct((NPAD, ND), _F32),
                   jax.ShapeDtypeStruct((NPAD, ND), _F32),
                   jax.ShapeDtypeStruct((G3, D), _F32)],
        scratch_shapes=[pltpu.VMEM((G3, ND), _F32),
                        pltpu.VMEM((G3, 1), _F32)],
    )(s2, cnt2, x, btc, w2e, b2e, wn1x, wn1a, bn1, wn2, bn2,
      wa, wb, wg1u, wg1n, bg1, wg2, bg2)


def _node1_body(s2, cnt2, x, btc, uin, w2e, b2e, wn1x, wn1a, bn1, wn2, bn2,
                wg1u, wg1n, bg1, wg2, bg2, u_ref, nacc, nb):
    i = pl.program_id(0)

    @pl.when(i == 0)
    def _():
        nacc[...] = jnp.zeros_like(nacc)
        nb[...] = jnp.zeros_like(nb)

    x2, m = _node_common(i, s2, cnt2, x, btc, w2e, b2e, wn1x, wn1a, bn1,
                         wn2, bn2)
    nacc[...] += m @ x2
    nb[...] += jnp.sum(m, axis=1, keepdims=True)

    @pl.when(i == NBLK - 1)
    def _():
        u_ref[...] = _glob_mlp(uin[...], nacc, nb, wg1u, wg1n, bg1, wg2, bg2)


def _node1(s2, cnt2, x, btc, uin, w2e, b2e, wn1x, wn1a, bn1, wn2, bn2,
           wg1u, wg1n, bg1, wg2, bg2):
    nblk = pl.BlockSpec((RB, ND), lambda i: (i, 0))
    return pl.pallas_call(
        _node1_body,
        grid=(NBLK,),
        in_specs=[pl.BlockSpec((1, 2, RB, ND),
                                lambda i: ((i % HBLK) // 6, 0, (i % HBLK) % 6, 0)),
                  pl.BlockSpec((1, 2, RB, ND),
                                lambda i: ((i % HBLK) // 6, 0, (i % HBLK) % 6, 0)),
                  nblk,
                  pl.BlockSpec((1, 1, RB), lambda i: (i, 0, 0)),
                  pl.BlockSpec((G3, D), lambda i: (0, 0)),
                  _full((D, D)), _full((1, D)),
                  _full((ND, ND)), _full((D, ND)), _full((1, ND)),
                  _full((ND, ND)), _full((1, ND)),
                  _full((D, D)), _full((ND, D)), _full((1, D)),
                  _full((D, D)), _full((1, D))],
        out_specs=pl.BlockSpec((G3, D), lambda i: (0, 0)),
        out_shape=jax.ShapeDtypeStruct((G3, D), _F32),
        scratch_shapes=[pltpu.VMEM((G3, ND), _F32),
                        pltpu.VMEM((G3, 1), _F32)],
    )(s2, cnt2, x, btc, uin, w2e, b2e, wn1x, wn1a, bn1, wn2, bn2,
      wg1u, wg1n, bg1, wg2, bg2)


def _p2_body(u, tc, y1, y2, ym,
             en1w, en1b, en2w, en2b, ee1w, ee1b, ee2w, ee2b,
             ew1, eb1, ew2, eb2, nw1, nb1, nw2, nb2,
             gw1, gb1, gw2, gb2, lw1, lb1, lw2, lb2, out_ref):
    x = (jnp.maximum(u[...] @ en1w[...] + en1b[...], 0.0) @ en2w[...]
         + en2b[...])
    t = tc[...]
    r1 = y1[...] / ym[...]
    r2 = y2[...] / ym[...]

    def enc_edge(r):
        h = jnp.maximum(t @ ee1w[0:1, :] + r @ ee1w[1:2, :] + ee1b[...], 0.0)
        return h @ ee2w[...] + ee2b[...]

    c1 = enc_edge(r1)
    c2 = enc_edge(r2)
    ea = jnp.concatenate([c1, c1, c2, c2], axis=0)
    ug = jnp.full((B, 32), 0.1, _F32)
    for _ in range(2):
        xa, xb, xc = x[0:B], x[B:2 * B], x[2 * B:3 * B]
        src = jnp.concatenate([xa, xc, xb, xc], axis=0)
        dst = jnp.concatenate([xc, xa, xc, xb], axis=0)
        cat = jnp.concatenate([src, dst, ea], axis=1)
        eh = jnp.maximum(cat @ ew1[...] + eb1[...], 0.0)
        ea = eh @ ew2[...] + eb2[...]
        agg = jnp.concatenate(
            [ea[B:2 * B], ea[3 * B:4 * B],
             (ea[0:B] + ea[2 * B:3 * B]) * 0.5], axis=0)
        ncat = jnp.concatenate([x, agg], axis=1)
        x = (jnp.maximum(ncat @ nw1[...] + nb1[...], 0.0) @ nw2[...]
             + nb2[...])
        xa, xb, xc = x[0:B], x[B:2 * B], x[2 * B:3 * B]
        nmean = (xa + xb + xc) * (1.0 / 3.0)
        gcat = jnp.concatenate([ug, nmean], axis=1)
        ug = (jnp.maximum(gcat @ gw1[...] + gb1[...], 0.0) @ gw2[...]
              + gb2[...])
    out_ref[...] = (jnp.maximum(ug @ lw1[...] + lb1[...], 0.0) @ lw2[...]
                    + lb2[...])


def _phase2(u, tc, y1, y2, ym, wlist):
    specs = [_full((G3, D))] + [_full((B, 1))] * 4
    specs += [_full(w.shape) for w in wlist]
    return pl.pallas_call(
        _p2_body,
        grid=(1,),
        in_specs=specs,
        out_specs=_full((B, 15)),
        out_shape=jax.ShapeDtypeStruct((B, 15), _F32),
    )(u, tc, y1, y2, ym, *wlist)


def kernel(x_p1, ei_p1, ea_p1, y_p1, btc_p1, x_p2, ei_p2, ea_p2, y_p2, btc_p2,
           x_pm, ei_pm, ea_pm, y_pm, btc_pm, Temperature, params):
    x_all = jnp.concatenate(
        [x_p1, x_p2, x_pm, jnp.zeros((NPAD - N, ND), _F32)], axis=0)
    row = jnp.concatenate([ei_p1[0], ei_p2[0] + NG, ei_pm[0] + 2 * NG,
                           jnp.zeros((EPAD - 3 * NE,), jnp.int32)])
    col = jnp.concatenate([ei_p1[1], ei_p2[1] + NG, ei_pm[1] + 2 * NG,
                           jnp.full((EPAD - 3 * NE,), N, jnp.int32)])
    row2 = row.reshape(NW, NCH, CH)
    col2 = col.reshape(NW, NCH, CH)
    colm = col % NH
    haH = NH // 2
    sidxA = jnp.where(colm < haH, colm, HTRASH).reshape(NW, NCH, CH)
    sidxB = jnp.where(colm >= haH, colm - haH, HTRASH).reshape(NW, NCH, CH)
    p0 = col < NH
    inA = colm < haH
    cA0 = jnp.where(p0 & inA, colm, HTRASH)
    cA1 = jnp.where(~p0 & inA, colm, HTRASH)
    cB0 = jnp.where(p0 & ~inA, colm - haH, HTRASH)
    cB1 = jnp.where(~p0 & ~inA, colm - haH, HTRASH)
    pkA = (cA0 | (cA1 << 13)).reshape(NW, NCH, CH)
    pkB = (cB0 | (cB1 << 13)).reshape(NW, NCH, CH)
    colv = col.reshape(EPAD, 1)
    ea_all = jnp.concatenate(
        [ea_p1, ea_p2, ea_pm, jnp.zeros((EPAD - 3 * NE, 4), _F32)], axis=0)
    btc = jnp.concatenate([btc_p1, btc_p2 + B, btc_pm + 2 * B,
                           jnp.full((NPAD - N,), G3, jnp.int32)])
    btc = btc.reshape(NBLK, 1, RB)
    zrow = jnp.zeros((CH, ND), _F32)
    o1 = jnp.ones((CH, D), _F32)
    o0 = jnp.zeros((CH, D), _F32)
    ones2 = jnp.stack([jnp.concatenate([o1, o0], axis=1),
                       jnp.concatenate([o0, o1], axis=1)])

    def wb(p):
        return [q for (W, bias) in p for q in (W, bias.reshape(1, -1))]

    en1w1, en1b1, en1w2, en1b2 = wb(params['enc_node_1'])
    ee1w1, ee1b1, ee1w2, ee1b2 = wb(params['enc_edge_1'])
    e1w1, e1b1, e1w2, e1b2 = wb(params['edge1'])
    w1a, w1b, w1c = e1w1[0:ND], e1w1[ND:2 * ND], e1w1[2 * ND:2 * ND + D]
    n1w1, n1b1, n1w2, n1b2 = wb(params['node1'])
    wn1x, wn1a = n1w1[0:ND], n1w1[ND:ND + D]
    g1w1, g1b1, g1w2, g1b2 = wb(params['glob1'])
    wg1u, wg1n = g1w1[0:D], g1w1[D:D + ND]
    p2w = (wb(params['enc_node_2']) + wb(params['enc_edge_2'])
           + wb(params['edge2']) + wb(params['node2'])
           + wb(params['glob2']) + wb(params['last']))

    ex, T = _encode(x_all, en1w1, en1b1, en1w2, en1b2, w1a, w1b)
    cnt2 = _sc_counts_call(pkA, pkB, zrow, ones2)

    gS = _sc_gather_call(T, row2, col2)
    hp0 = _edge0(gS, ea_all, colv, ee1w1, ee1b1, ee1w2, ee1b2, w1c, e1b1)
    S2 = _sc_scatter_call(hp0, sidxA, sidxB, zrow)
    x2, T2, u1 = _node0(S2, cnt2, ex, btc, e1w2, e1b2,
                        wn1x, wn1a, n1b1, n1w2, n1b2, w1a, w1b,
                        wg1u, wg1n, g1b1, g1w2, g1b2)

    gS = _sc_gather_call(T2, row2, col2)
    hp1 = _edge1(gS, hp0, colv, w1c, e1b1, e1w2, e1b2)
    S2 = _sc_scatter_call(hp1, sidxA, sidxB, zrow)
    u2 = _node1(S2, cnt2, x2, btc, u1, e1w2, e1b2,
                wn1x, wn1a, n1b1, n1w2, n1b2,
                wg1u, wg1n, g1b1, g1w2, g1b2)

    return _phase2(u2, Temperature.reshape(B, 1), y_p1.reshape(B, 1),
                   y_p2.reshape(B, 1), y_pm.reshape(B, 1), p2w)
